# baseline (speedup 1.0000x reference)
import functools

import jax
import jax.numpy as jnp
from jax import lax
from jax.experimental import pallas as pl
from jax.experimental.pallas import tpu as pltpu
from jax.experimental.pallas import tpu_sc as plsc

N = 10000
E = 320000
NODE_DIM = 128
EDGE_DIM = 16
HIDDEN = 128
LAYERS = 2
OUT_DIM = 128
HEADS = 4
HEAD_DIM = HIDDEN // HEADS
G = 16

_LANES = 16
_CH = 128
_NCHUNKS = E // _CH
_NW = 32
_CPW = _NCHUNKS // _NW
_REM = _NCHUNKS - _CPW * _NW
_RPB = 632
_RPB_LAST = N - 15 * _RPB


def _mlp_body(x_ref, w1_ref, b1_ref, w2_ref, b2_ref, o_ref):
    t = jnp.maximum(x_ref[...] @ w1_ref[...] + b1_ref[...], 0.0)
    o_ref[...] = t @ w2_ref[...] + b2_ref[...]


def _mlp(x, p, rows):
    n, din = x.shape
    dh = p["l1"]["W"].shape[1]
    dout = p["l2"]["W"].shape[1]
    return pl.pallas_call(
        _mlp_body,
        grid=(n // rows,),
        in_specs=[
            pl.BlockSpec((rows, din), lambda i: (i, 0)),
            pl.BlockSpec((din, dh), lambda i: (0, 0)),
            pl.BlockSpec((1, dh), lambda i: (0, 0)),
            pl.BlockSpec((dh, dout), lambda i: (0, 0)),
            pl.BlockSpec((1, dout), lambda i: (0, 0)),
        ],
        out_specs=pl.BlockSpec((rows, dout), lambda i: (i, 0)),
        out_shape=jax.ShapeDtypeStruct((n, dout), jnp.float32),
    )(x, p["l1"]["W"], p["l1"]["b"].reshape(1, dh),
      p["l2"]["W"], p["l2"]["b"].reshape(1, dout))


def _conv_body(h_ref, a0_ref, a1_ref, w1_ref, b1_ref, w2_ref, b2_ref,
               g_ref, bn_ref, wm_ref, bm_ref, hn_ref, ms_ref):
    h = h_ref[...]
    a = h + a0_ref[0] + a1_ref[0]
    t = jnp.maximum(a @ w1_ref[...] + b1_ref[...], 0.0)
    o = jnp.maximum(t @ w2_ref[...] + b2_ref[...], 0.0)
    o = o + h
    m = jnp.mean(o, axis=-1, keepdims=True)
    v = jnp.mean((o - m) ** 2, axis=-1, keepdims=True)
    hn = (o - m) / jnp.sqrt(v + 1e-5) * g_ref[...] + bn_ref[...]
    hn_ref[...] = hn
    ms_ref[...] = hn @ wm_ref[...] + bm_ref[...]


def _conv(h, aggr2, cp, np_, mp, rows):
    full = lambda shape: pl.BlockSpec(shape, lambda i: (0, 0))
    row_spec = pl.BlockSpec((rows, HIDDEN), lambda i: (i, 0))
    hn, ms = pl.pallas_call(
        _conv_body,
        grid=(N // rows,),
        in_specs=[
            row_spec,
            pl.BlockSpec((1, rows, HIDDEN), lambda i: (0, i, 0)),
            pl.BlockSpec((1, rows, HIDDEN), lambda i: (1, i, 0)),
            full((HIDDEN, HIDDEN)), full((1, HIDDEN)),
            full((HIDDEN, HIDDEN)), full((1, HIDDEN)),
            full((1, HIDDEN)), full((1, HIDDEN)),
            full((HIDDEN, HIDDEN)), full((1, HIDDEN)),
        ],
        out_specs=[row_spec, row_spec],
        out_shape=[jax.ShapeDtypeStruct((N, HIDDEN), jnp.float32),
                   jax.ShapeDtypeStruct((N, HIDDEN), jnp.float32)],
    )(h, aggr2, aggr2,
      cp["l1"]["W"], cp["l1"]["b"].reshape(1, HIDDEN),
      cp["l2"]["W"], cp["l2"]["b"].reshape(1, HIDDEN),
      np_["g"].reshape(1, HIDDEN), np_["b"].reshape(1, HIDDEN),
      mp["W"], mp["b"].reshape(1, HIDDEN))
    return hn, ms


_CONTRACT0 = (((0,), (0,)), ((), ()))


def _onehot(bid):
    return (bid == lax.broadcasted_iota(jnp.int32, (1, G), 1)
            ).astype(jnp.float32)


def _scores_body(h_ref, wk_ref, bk_ref, qm_ref, s_ref):
    k = h_ref[...] @ wk_ref[...] + bk_ref[...]
    s_ref[...] = (k @ qm_ref[...]) * (1.0 / jnp.sqrt(jnp.float32(HEAD_DIM)))


def _softmax_body(s_ref, bid_ref, a_ref):
    scores = s_ref[...].T
    p_t = (bid_ref[...] == lax.broadcasted_iota(jnp.int32, (G, 1), 0)
           ).astype(jnp.float32)
    smax_cols = []
    for g in range(G):
        mask = p_t[g:g + 1, :] > 0.0
        sg = jnp.max(jnp.where(mask, scores, -jnp.inf), axis=1, keepdims=True)
        smax_cols.append(sg)
    smax = jnp.concatenate(smax_cols, axis=1)
    smax = jnp.where(smax > -1e30, smax, 0.0)
    e = jnp.exp(scores - smax @ p_t)
    contract1 = (((1,), (1,)), ((), ()))
    denom = lax.dot_general(e, p_t, contract1)
    a_ref[...] = (e / jnp.maximum(denom @ p_t, 1e-12)).T


def _pool_body(h_ref, ms1_ref, ms2_ref, attn_ref, bid_ref,
               wv_ref, bv_ref, r4_ref, gemb_ref, p1_ref, p2_ref):
    i = pl.program_id(0)
    p_onehot = _onehot(bid_ref[...])
    v = h_ref[...] @ wv_ref[...] + bv_ref[...]
    wvw = (attn_ref[...] @ r4_ref[...]) * v

    @pl.when(i == 0)
    def _():
        gemb_ref[...] = jnp.zeros_like(gemb_ref)
        p1_ref[...] = jnp.zeros_like(p1_ref)
        p2_ref[...] = jnp.zeros_like(p2_ref)

    gemb_ref[...] += lax.dot_general(p_onehot, wvw, _CONTRACT0)
    p1_ref[...] += lax.dot_general(p_onehot, ms1_ref[...], _CONTRACT0)
    p2_ref[...] += lax.dot_general(p_onehot, ms2_ref[...], _CONTRACT0)


def _head_body(gemb_ref, p1_ref, p2_ref, wo_ref, bo_ref, ng_ref, nb_ref,
               w1_ref, b1_ref, w2_ref, b2_ref, w3_ref, b3_ref, o_ref):
    gemb = gemb_ref[...] @ wo_ref[...] + bo_ref[...]
    m = jnp.mean(gemb, axis=-1, keepdims=True)
    var = jnp.mean((gemb - m) ** 2, axis=-1, keepdims=True)
    gemb = (gemb - m) / jnp.sqrt(var + 1e-5) * ng_ref[...] + nb_ref[...]
    cat = jnp.concatenate([gemb, p1_ref[...], p2_ref[...]], axis=-1)
    t = jnp.maximum(cat @ w1_ref[...] + b1_ref[...], 0.0)
    t = jnp.maximum(t @ w2_ref[...] + b2_ref[...], 0.0)
    p_out = t @ w3_ref[...] + b3_ref[...]
    nrm = jnp.sqrt(jnp.sum(p_out * p_out, axis=-1, keepdims=True))
    o_ref[...] = p_out / jnp.maximum(nrm, 1e-12)


def _readout(h, ms1, ms2, batch_ids, rp, hp, rows=2000):
    q = rp["query"].reshape(HEADS, HEAD_DIM)
    eye = jnp.eye(HEADS, dtype=jnp.float32)
    qmat = (q[:, :, None] * eye[:, None, :]).reshape(HIDDEN, HEADS)
    r4 = jnp.repeat(eye, HEAD_DIM, axis=1)
    bid = batch_ids.reshape(N, 1).astype(jnp.int32)
    bid_t = batch_ids.reshape(1, N).astype(jnp.int32)
    full = lambda shape: pl.BlockSpec(shape, lambda i: (0, 0))
    row_h = pl.BlockSpec((rows, HIDDEN), lambda i: (i, 0))
    row_s = pl.BlockSpec((rows, HEADS), lambda i: (i, 0))
    row_b = pl.BlockSpec((rows, 1), lambda i: (i, 0))

    scores = pl.pallas_call(
        _scores_body,
        grid=(N // rows,),
        in_specs=[row_h, full((HIDDEN, HIDDEN)), full((1, HIDDEN)),
                  full((HIDDEN, HEADS))],
        out_specs=row_s,
        out_shape=jax.ShapeDtypeStruct((N, HEADS), jnp.float32),
    )(h, rp["key"]["W"], rp["key"]["b"].reshape(1, HIDDEN), qmat)

    attn = pl.pallas_call(
        _softmax_body,
        out_shape=jax.ShapeDtypeStruct((N, HEADS), jnp.float32),
    )(scores, bid_t)

    gspec = pl.BlockSpec((G, HIDDEN), lambda i: (0, 0))
    gemb, p1, p2 = pl.pallas_call(
        _pool_body,
        grid=(N // rows,),
        in_specs=[row_h, row_h, row_h, row_s, row_b,
                  full((HIDDEN, HIDDEN)), full((1, HIDDEN)),
                  full((HEADS, HIDDEN))],
        out_specs=[gspec, gspec, gspec],
        out_shape=[jax.ShapeDtypeStruct((G, HIDDEN), jnp.float32)] * 3,
    )(h, ms1, ms2, attn, bid,
      rp["value"]["W"], rp["value"]["b"].reshape(1, HIDDEN), r4)

    return pl.pallas_call(
        _head_body,
        out_shape=jax.ShapeDtypeStruct((G, OUT_DIM), jnp.float32),
    )(gemb, p1, p2,
      rp["out"]["W"], rp["out"]["b"].reshape(1, HIDDEN),
      rp["ng"].reshape(1, HIDDEN), rp["nb"].reshape(1, HIDDEN),
      hp["l1"]["W"], hp["l1"]["b"].reshape(1, HIDDEN),
      hp["l2"]["W"], hp["l2"]["b"].reshape(1, HIDDEN),
      hp["l3"]["W"], hp["l3"]["b"].reshape(1, OUT_DIM))


def _gine_aggr(h, ea, src, dst):
    mesh = plsc.VectorSubcoreMesh(core_axis_name="c", subcore_axis_name="s")

    @functools.partial(
        pl.kernel, mesh=mesh,
        out_type=jax.ShapeDtypeStruct((2, N, HIDDEN), jnp.float32),
        scratch_types=[
            pltpu.VMEM((_CH,), jnp.int32),
            pltpu.VMEM((_CH,), jnp.int32),
            pltpu.VMEM((_CH, HIDDEN), jnp.float32),
            pltpu.VMEM((_CH, HIDDEN), jnp.float32),
            pltpu.VMEM_SHARED((N, HIDDEN), jnp.float32),
            pltpu.SemaphoreType.DMA,
        ],
    )
    def k(h_hbm, ea_hbm, src_hbm, dst_hbm, z_hbm, out_hbm,
          srcv, dstv, rowsv, eav, aggr_sh, sem):
        c = lax.axis_index("c")
        s = lax.axis_index("s")
        wid = s * 2 + c
        roff = pl.multiple_of(s * _RPB, 8)
        @pl.when(s < 15)
        def _():
            pltpu.sync_copy(z_hbm.at[pl.ds(roff, _RPB)],
                            aggr_sh.at[pl.ds(roff, _RPB)])

        @pl.when(s == 15)
        def _():
            pltpu.sync_copy(z_hbm.at[pl.ds(roff, _RPB_LAST)],
                            aggr_sh.at[pl.ds(roff, _RPB_LAST)])

        plsc.subcore_barrier()
        nch = jnp.where(wid < _REM, _CPW + 1, _CPW)
        cbase = wid * _CPW + jnp.minimum(wid, _REM)

        def chunk(kk, carry):
            eoff = pl.multiple_of((cbase + kk) * _CH, _CH)
            pltpu.sync_copy(src_hbm.at[pl.ds(eoff, _CH)], srcv)
            pltpu.sync_copy(dst_hbm.at[pl.ds(eoff, _CH)], dstv)
            cp = pltpu.async_copy(h_hbm.at[srcv], rowsv, sem)
            pltpu.sync_copy(ea_hbm.at[pl.ds(eoff, _CH)], eav)
            cp.wait()

            def rbody(i, cr):
                for j in range(HIDDEN // _LANES):
                    sl = pl.ds(j * _LANES, _LANES)
                    rowsv[i, sl] = jnp.maximum(rowsv[i, sl] + eav[i, sl], 0.0)
                return cr

            lax.fori_loop(0, _CH, rbody, 0, unroll=4)
            pltpu.sync_copy(rowsv, aggr_sh.at[dstv], add=True)
            return carry

        lax.fori_loop(0, nch, chunk, 0)
        plsc.subcore_barrier()

        @pl.when(s < 15)
        def _():
            pltpu.sync_copy(aggr_sh.at[pl.ds(roff, _RPB)],
                            out_hbm.at[c, pl.ds(roff, _RPB)])

        @pl.when(s == 15)
        def _():
            pltpu.sync_copy(aggr_sh.at[pl.ds(roff, _RPB_LAST)],
                            out_hbm.at[c, pl.ds(roff, _RPB_LAST)])

    return k(h, ea, src, dst, jnp.zeros((N, HIDDEN), jnp.float32))


def kernel(x, edge_attr, params, edge_index, batch_ids):
    p = params
    h = _mlp(x, p["node_enc"], rows=2000)
    ea = _mlp(edge_attr, p["edge_enc"], rows=4000)
    src = edge_index[0]
    dst = edge_index[1]
    ms = []
    for i in range(LAYERS):
        aggr2 = _gine_aggr(h, ea, src, dst)
        h, m = _conv(h, aggr2, p["convs"][i], p["norms"][i], p["ms"][i],
                     rows=2000)
        ms.append(m)
    return _readout(h, ms[0], ms[1], batch_ids, p["readout"], p["head"])


# speedup vs baseline: 1.5934x; 1.5934x over previous
import functools

import jax
import jax.numpy as jnp
from jax import lax
from jax.experimental import pallas as pl
from jax.experimental.pallas import tpu as pltpu
from jax.experimental.pallas import tpu_sc as plsc

N = 10000
E = 320000
NODE_DIM = 128
EDGE_DIM = 16
HIDDEN = 128
LAYERS = 2
OUT_DIM = 128
HEADS = 4
HEAD_DIM = HIDDEN // HEADS
G = 16

_LANES = 16
_CH = 128
_NCHUNKS = E // _CH
_NW = 32
_CPW = _NCHUNKS // _NW
_REM = _NCHUNKS - _CPW * _NW
_RPB = 632
_RPB_LAST = N - 15 * _RPB


def _mlp_body(x_ref, w1_ref, b1_ref, w2_ref, b2_ref, o_ref):
    t = jnp.maximum(x_ref[...] @ w1_ref[...] + b1_ref[...], 0.0)
    o_ref[...] = t @ w2_ref[...] + b2_ref[...]


def _mlp(x, p, rows):
    n, din = x.shape
    dh = p["l1"]["W"].shape[1]
    dout = p["l2"]["W"].shape[1]
    return pl.pallas_call(
        _mlp_body,
        grid=(n // rows,),
        in_specs=[
            pl.BlockSpec((rows, din), lambda i: (i, 0)),
            pl.BlockSpec((din, dh), lambda i: (0, 0)),
            pl.BlockSpec((1, dh), lambda i: (0, 0)),
            pl.BlockSpec((dh, dout), lambda i: (0, 0)),
            pl.BlockSpec((1, dout), lambda i: (0, 0)),
        ],
        out_specs=pl.BlockSpec((rows, dout), lambda i: (i, 0)),
        out_shape=jax.ShapeDtypeStruct((n, dout), jnp.float32),
    )(x, p["l1"]["W"], p["l1"]["b"].reshape(1, dh),
      p["l2"]["W"], p["l2"]["b"].reshape(1, dout))


def _conv_body(h_ref, a0_ref, a1_ref, w1_ref, b1_ref, w2_ref, b2_ref,
               g_ref, bn_ref, wm_ref, bm_ref, hn_ref, ms_ref):
    h = h_ref[...]
    a = h + a0_ref[0] + a1_ref[0]
    t = jnp.maximum(a @ w1_ref[...] + b1_ref[...], 0.0)
    o = jnp.maximum(t @ w2_ref[...] + b2_ref[...], 0.0)
    o = o + h
    m = jnp.mean(o, axis=-1, keepdims=True)
    v = jnp.mean((o - m) ** 2, axis=-1, keepdims=True)
    hn = (o - m) / jnp.sqrt(v + 1e-5) * g_ref[...] + bn_ref[...]
    hn_ref[...] = hn
    ms_ref[...] = hn @ wm_ref[...] + bm_ref[...]


def _conv(h, aggr2, cp, np_, mp, rows):
    full = lambda shape: pl.BlockSpec(shape, lambda i: (0, 0))
    row_spec = pl.BlockSpec((rows, HIDDEN), lambda i: (i, 0))
    hn, ms = pl.pallas_call(
        _conv_body,
        grid=(N // rows,),
        in_specs=[
            row_spec,
            pl.BlockSpec((1, rows, HIDDEN), lambda i: (0, i, 0)),
            pl.BlockSpec((1, rows, HIDDEN), lambda i: (1, i, 0)),
            full((HIDDEN, HIDDEN)), full((1, HIDDEN)),
            full((HIDDEN, HIDDEN)), full((1, HIDDEN)),
            full((1, HIDDEN)), full((1, HIDDEN)),
            full((HIDDEN, HIDDEN)), full((1, HIDDEN)),
        ],
        out_specs=[row_spec, row_spec],
        out_shape=[jax.ShapeDtypeStruct((N, HIDDEN), jnp.float32),
                   jax.ShapeDtypeStruct((N, HIDDEN), jnp.float32)],
    )(h, aggr2, aggr2,
      cp["l1"]["W"], cp["l1"]["b"].reshape(1, HIDDEN),
      cp["l2"]["W"], cp["l2"]["b"].reshape(1, HIDDEN),
      np_["g"].reshape(1, HIDDEN), np_["b"].reshape(1, HIDDEN),
      mp["W"], mp["b"].reshape(1, HIDDEN))
    return hn, ms


_CONTRACT0 = (((0,), (0,)), ((), ()))


def _onehot(bid):
    return (bid == lax.broadcasted_iota(jnp.int32, (1, G), 1)
            ).astype(jnp.float32)


def _scores_body(h_ref, wk_ref, bk_ref, qm_ref, s_ref):
    k = h_ref[...] @ wk_ref[...] + bk_ref[...]
    s_ref[...] = (k @ qm_ref[...]) * (1.0 / jnp.sqrt(jnp.float32(HEAD_DIM)))


def _softmax_body(s_ref, bid_ref, a_ref):
    scores = s_ref[...].T
    p_t = (bid_ref[...] == lax.broadcasted_iota(jnp.int32, (G, 1), 0)
           ).astype(jnp.float32)
    smax_cols = []
    for g in range(G):
        mask = p_t[g:g + 1, :] > 0.0
        sg = jnp.max(jnp.where(mask, scores, -jnp.inf), axis=1, keepdims=True)
        smax_cols.append(sg)
    smax = jnp.concatenate(smax_cols, axis=1)
    smax = jnp.where(smax > -1e30, smax, 0.0)
    e = jnp.exp(scores - smax @ p_t)
    contract1 = (((1,), (1,)), ((), ()))
    denom = lax.dot_general(e, p_t, contract1)
    a_ref[...] = (e / jnp.maximum(denom @ p_t, 1e-12)).T


def _pool_body(h_ref, ms1_ref, ms2_ref, attn_ref, bid_ref,
               wv_ref, bv_ref, r4_ref, gemb_ref, p1_ref, p2_ref):
    i = pl.program_id(0)
    p_onehot = _onehot(bid_ref[...])
    v = h_ref[...] @ wv_ref[...] + bv_ref[...]
    wvw = (attn_ref[...] @ r4_ref[...]) * v

    @pl.when(i == 0)
    def _():
        gemb_ref[...] = jnp.zeros_like(gemb_ref)
        p1_ref[...] = jnp.zeros_like(p1_ref)
        p2_ref[...] = jnp.zeros_like(p2_ref)

    gemb_ref[...] += lax.dot_general(p_onehot, wvw, _CONTRACT0)
    p1_ref[...] += lax.dot_general(p_onehot, ms1_ref[...], _CONTRACT0)
    p2_ref[...] += lax.dot_general(p_onehot, ms2_ref[...], _CONTRACT0)


def _head_body(gemb_ref, p1_ref, p2_ref, wo_ref, bo_ref, ng_ref, nb_ref,
               w1_ref, b1_ref, w2_ref, b2_ref, w3_ref, b3_ref, o_ref):
    gemb = gemb_ref[...] @ wo_ref[...] + bo_ref[...]
    m = jnp.mean(gemb, axis=-1, keepdims=True)
    var = jnp.mean((gemb - m) ** 2, axis=-1, keepdims=True)
    gemb = (gemb - m) / jnp.sqrt(var + 1e-5) * ng_ref[...] + nb_ref[...]
    cat = jnp.concatenate([gemb, p1_ref[...], p2_ref[...]], axis=-1)
    t = jnp.maximum(cat @ w1_ref[...] + b1_ref[...], 0.0)
    t = jnp.maximum(t @ w2_ref[...] + b2_ref[...], 0.0)
    p_out = t @ w3_ref[...] + b3_ref[...]
    nrm = jnp.sqrt(jnp.sum(p_out * p_out, axis=-1, keepdims=True))
    o_ref[...] = p_out / jnp.maximum(nrm, 1e-12)


def _readout(h, ms1, ms2, batch_ids, rp, hp, rows=2000):
    q = rp["query"].reshape(HEADS, HEAD_DIM)
    eye = jnp.eye(HEADS, dtype=jnp.float32)
    qmat = (q[:, :, None] * eye[:, None, :]).reshape(HIDDEN, HEADS)
    r4 = jnp.repeat(eye, HEAD_DIM, axis=1)
    bid = batch_ids.reshape(N, 1).astype(jnp.int32)
    bid_t = batch_ids.reshape(1, N).astype(jnp.int32)
    full = lambda shape: pl.BlockSpec(shape, lambda i: (0, 0))
    row_h = pl.BlockSpec((rows, HIDDEN), lambda i: (i, 0))
    row_s = pl.BlockSpec((rows, HEADS), lambda i: (i, 0))
    row_b = pl.BlockSpec((rows, 1), lambda i: (i, 0))

    scores = pl.pallas_call(
        _scores_body,
        grid=(N // rows,),
        in_specs=[row_h, full((HIDDEN, HIDDEN)), full((1, HIDDEN)),
                  full((HIDDEN, HEADS))],
        out_specs=row_s,
        out_shape=jax.ShapeDtypeStruct((N, HEADS), jnp.float32),
    )(h, rp["key"]["W"], rp["key"]["b"].reshape(1, HIDDEN), qmat)

    attn = pl.pallas_call(
        _softmax_body,
        out_shape=jax.ShapeDtypeStruct((N, HEADS), jnp.float32),
    )(scores, bid_t)

    gspec = pl.BlockSpec((G, HIDDEN), lambda i: (0, 0))
    gemb, p1, p2 = pl.pallas_call(
        _pool_body,
        grid=(N // rows,),
        in_specs=[row_h, row_h, row_h, row_s, row_b,
                  full((HIDDEN, HIDDEN)), full((1, HIDDEN)),
                  full((HEADS, HIDDEN))],
        out_specs=[gspec, gspec, gspec],
        out_shape=[jax.ShapeDtypeStruct((G, HIDDEN), jnp.float32)] * 3,
    )(h, ms1, ms2, attn, bid,
      rp["value"]["W"], rp["value"]["b"].reshape(1, HIDDEN), r4)

    return pl.pallas_call(
        _head_body,
        out_shape=jax.ShapeDtypeStruct((G, OUT_DIM), jnp.float32),
    )(gemb, p1, p2,
      rp["out"]["W"], rp["out"]["b"].reshape(1, HIDDEN),
      rp["ng"].reshape(1, HIDDEN), rp["nb"].reshape(1, HIDDEN),
      hp["l1"]["W"], hp["l1"]["b"].reshape(1, HIDDEN),
      hp["l2"]["W"], hp["l2"]["b"].reshape(1, HIDDEN),
      hp["l3"]["W"], hp["l3"]["b"].reshape(1, OUT_DIM))


def _gine_aggr(h, ea, src, dst):
    mesh = plsc.VectorSubcoreMesh(core_axis_name="c", subcore_axis_name="s")

    @functools.partial(
        pl.kernel, mesh=mesh,
        out_type=jax.ShapeDtypeStruct((2, N, HIDDEN), jnp.float32),
        scratch_types=[
            pltpu.VMEM((_CH,), jnp.int32),
            pltpu.VMEM((_CH,), jnp.int32),
            pltpu.VMEM((_CH, HIDDEN), jnp.float32),
            pltpu.VMEM((_CH, HIDDEN), jnp.float32),
            pltpu.VMEM_SHARED((N, HIDDEN), jnp.float32),
            pltpu.SemaphoreType.DMA,
        ],
    )
    def k(h_hbm, ea_hbm, src_hbm, dst_hbm, z_hbm, out_hbm,
          srcv, dstv, rowsv, eav, aggr_sh, sem):
        c = lax.axis_index("c")
        s = lax.axis_index("s")
        wid = s * 2 + c
        roff = pl.multiple_of(s * _RPB, 8)
        @pl.when(s < 15)
        def _():
            pltpu.sync_copy(z_hbm.at[pl.ds(roff, _RPB)],
                            aggr_sh.at[pl.ds(roff, _RPB)])

        @pl.when(s == 15)
        def _():
            pltpu.sync_copy(z_hbm.at[pl.ds(roff, _RPB_LAST)],
                            aggr_sh.at[pl.ds(roff, _RPB_LAST)])

        plsc.subcore_barrier()
        nch = jnp.where(wid < _REM, _CPW + 1, _CPW)
        cbase = wid * _CPW + jnp.minimum(wid, _REM)

        def chunk(kk, carry):
            eoff = pl.multiple_of((cbase + kk) * _CH, _CH)
            pltpu.sync_copy(src_hbm.at[pl.ds(eoff, _CH)], srcv)
            pltpu.sync_copy(dst_hbm.at[pl.ds(eoff, _CH)], dstv)
            cp = pltpu.async_copy(h_hbm.at[srcv], rowsv, sem)
            pltpu.sync_copy(ea_hbm.at[pl.ds(eoff, _CH)], eav)
            cp.wait()

            def rbody(i, cr):
                for j in range(HIDDEN // _LANES):
                    sl = pl.ds(j * _LANES, _LANES)
                    rowsv[i, sl] = jnp.maximum(rowsv[i, sl] + eav[i, sl], 0.0)
                return cr

            lax.fori_loop(0, _CH, rbody, 0)
            pltpu.sync_copy(rowsv, aggr_sh.at[dstv], add=True)
            return carry

        lax.fori_loop(0, nch, chunk, 0)
        plsc.subcore_barrier()

        @pl.when(s < 15)
        def _():
            pltpu.sync_copy(aggr_sh.at[pl.ds(roff, _RPB)],
                            out_hbm.at[c, pl.ds(roff, _RPB)])

        @pl.when(s == 15)
        def _():
            pltpu.sync_copy(aggr_sh.at[pl.ds(roff, _RPB_LAST)],
                            out_hbm.at[c, pl.ds(roff, _RPB_LAST)])

    return k(h, ea, src, dst, jnp.zeros((N, HIDDEN), jnp.float32))


def kernel(x, edge_attr, params, edge_index, batch_ids):
    p = params
    h = _mlp(x, p["node_enc"], rows=2000)
    ea = _mlp(edge_attr, p["edge_enc"], rows=4000)
    src = edge_index[0]
    dst = edge_index[1]
    ms = []
    for i in range(LAYERS):
        aggr2 = _gine_aggr(h, ea, src, dst)
        h, m = _conv(h, aggr2, p["convs"][i], p["norms"][i], p["ms"][i],
                     rows=2000)
        ms.append(m)
    return _readout(h, ms[0], ms[1], batch_ids, p["readout"], p["head"])


# speedup vs baseline: 1.7030x; 1.0687x over previous
import functools

import jax
import jax.numpy as jnp
from jax import lax
from jax.experimental import pallas as pl
from jax.experimental.pallas import tpu as pltpu
from jax.experimental.pallas import tpu_sc as plsc

N = 10000
E = 320000
NODE_DIM = 128
EDGE_DIM = 16
HIDDEN = 128
LAYERS = 2
OUT_DIM = 128
HEADS = 4
HEAD_DIM = HIDDEN // HEADS
G = 16

_LANES = 16
_CH = 128
_NCHUNKS = E // _CH
_NW = 32
_CPW = _NCHUNKS // _NW
_REM = _NCHUNKS - _CPW * _NW
_RPB = 632
_RPB_LAST = N - 15 * _RPB


def _mlp_body(x_ref, w1_ref, b1_ref, w2_ref, b2_ref, o_ref):
    t = jnp.maximum(x_ref[...] @ w1_ref[...] + b1_ref[...], 0.0)
    o_ref[...] = t @ w2_ref[...] + b2_ref[...]


def _mlp(x, p, rows):
    n, din = x.shape
    dh = p["l1"]["W"].shape[1]
    dout = p["l2"]["W"].shape[1]
    return pl.pallas_call(
        _mlp_body,
        grid=(n // rows,),
        in_specs=[
            pl.BlockSpec((rows, din), lambda i: (i, 0)),
            pl.BlockSpec((din, dh), lambda i: (0, 0)),
            pl.BlockSpec((1, dh), lambda i: (0, 0)),
            pl.BlockSpec((dh, dout), lambda i: (0, 0)),
            pl.BlockSpec((1, dout), lambda i: (0, 0)),
        ],
        out_specs=pl.BlockSpec((rows, dout), lambda i: (i, 0)),
        out_shape=jax.ShapeDtypeStruct((n, dout), jnp.float32),
    )(x, p["l1"]["W"], p["l1"]["b"].reshape(1, dh),
      p["l2"]["W"], p["l2"]["b"].reshape(1, dout))


def _conv_body(h_ref, a0_ref, a1_ref, w1_ref, b1_ref, w2_ref, b2_ref,
               g_ref, bn_ref, wm_ref, bm_ref, hn_ref, ms_ref):
    h = h_ref[...]
    a = h + a0_ref[0] + a1_ref[0]
    t = jnp.maximum(a @ w1_ref[...] + b1_ref[...], 0.0)
    o = jnp.maximum(t @ w2_ref[...] + b2_ref[...], 0.0)
    o = o + h
    m = jnp.mean(o, axis=-1, keepdims=True)
    v = jnp.mean((o - m) ** 2, axis=-1, keepdims=True)
    hn = (o - m) / jnp.sqrt(v + 1e-5) * g_ref[...] + bn_ref[...]
    hn_ref[...] = hn
    ms_ref[...] = hn @ wm_ref[...] + bm_ref[...]


def _conv(h, aggr2, cp, np_, mp, rows):
    full = lambda shape: pl.BlockSpec(shape, lambda i: (0, 0))
    row_spec = pl.BlockSpec((rows, HIDDEN), lambda i: (i, 0))
    hn, ms = pl.pallas_call(
        _conv_body,
        grid=(N // rows,),
        in_specs=[
            row_spec,
            pl.BlockSpec((1, rows, HIDDEN), lambda i: (0, i, 0)),
            pl.BlockSpec((1, rows, HIDDEN), lambda i: (1, i, 0)),
            full((HIDDEN, HIDDEN)), full((1, HIDDEN)),
            full((HIDDEN, HIDDEN)), full((1, HIDDEN)),
            full((1, HIDDEN)), full((1, HIDDEN)),
            full((HIDDEN, HIDDEN)), full((1, HIDDEN)),
        ],
        out_specs=[row_spec, row_spec],
        out_shape=[jax.ShapeDtypeStruct((N, HIDDEN), jnp.float32),
                   jax.ShapeDtypeStruct((N, HIDDEN), jnp.float32)],
    )(h, aggr2, aggr2,
      cp["l1"]["W"], cp["l1"]["b"].reshape(1, HIDDEN),
      cp["l2"]["W"], cp["l2"]["b"].reshape(1, HIDDEN),
      np_["g"].reshape(1, HIDDEN), np_["b"].reshape(1, HIDDEN),
      mp["W"], mp["b"].reshape(1, HIDDEN))
    return hn, ms


_CONTRACT0 = (((0,), (0,)), ((), ()))


def _onehot(bid):
    return (bid == lax.broadcasted_iota(jnp.int32, (1, G), 1)
            ).astype(jnp.float32)


def _scores_body(h_ref, wk_ref, bk_ref, qm_ref, s_ref):
    k = h_ref[...] @ wk_ref[...] + bk_ref[...]
    s_ref[...] = (k @ qm_ref[...]) * (1.0 / jnp.sqrt(jnp.float32(HEAD_DIM)))


def _softmax_body(s_ref, bid_ref, a_ref):
    scores = s_ref[...].T
    p_t = (bid_ref[...] == lax.broadcasted_iota(jnp.int32, (G, 1), 0)
           ).astype(jnp.float32)
    smax_cols = []
    for g in range(G):
        mask = p_t[g:g + 1, :] > 0.0
        sg = jnp.max(jnp.where(mask, scores, -jnp.inf), axis=1, keepdims=True)
        smax_cols.append(sg)
    smax = jnp.concatenate(smax_cols, axis=1)
    smax = jnp.where(smax > -1e30, smax, 0.0)
    e = jnp.exp(scores - smax @ p_t)
    contract1 = (((1,), (1,)), ((), ()))
    denom = lax.dot_general(e, p_t, contract1)
    a_ref[...] = (e / jnp.maximum(denom @ p_t, 1e-12)).T


def _pool_body(h_ref, ms1_ref, ms2_ref, attn_ref, bid_ref,
               wv_ref, bv_ref, r4_ref, gemb_ref, p1_ref, p2_ref):
    i = pl.program_id(0)
    p_onehot = _onehot(bid_ref[...])
    v = h_ref[...] @ wv_ref[...] + bv_ref[...]
    wvw = (attn_ref[...] @ r4_ref[...]) * v

    @pl.when(i == 0)
    def _():
        gemb_ref[...] = jnp.zeros_like(gemb_ref)
        p1_ref[...] = jnp.zeros_like(p1_ref)
        p2_ref[...] = jnp.zeros_like(p2_ref)

    gemb_ref[...] += lax.dot_general(p_onehot, wvw, _CONTRACT0)
    p1_ref[...] += lax.dot_general(p_onehot, ms1_ref[...], _CONTRACT0)
    p2_ref[...] += lax.dot_general(p_onehot, ms2_ref[...], _CONTRACT0)


def _head_body(gemb_ref, p1_ref, p2_ref, wo_ref, bo_ref, ng_ref, nb_ref,
               w1_ref, b1_ref, w2_ref, b2_ref, w3_ref, b3_ref, o_ref):
    gemb = gemb_ref[...] @ wo_ref[...] + bo_ref[...]
    m = jnp.mean(gemb, axis=-1, keepdims=True)
    var = jnp.mean((gemb - m) ** 2, axis=-1, keepdims=True)
    gemb = (gemb - m) / jnp.sqrt(var + 1e-5) * ng_ref[...] + nb_ref[...]
    cat = jnp.concatenate([gemb, p1_ref[...], p2_ref[...]], axis=-1)
    t = jnp.maximum(cat @ w1_ref[...] + b1_ref[...], 0.0)
    t = jnp.maximum(t @ w2_ref[...] + b2_ref[...], 0.0)
    p_out = t @ w3_ref[...] + b3_ref[...]
    nrm = jnp.sqrt(jnp.sum(p_out * p_out, axis=-1, keepdims=True))
    o_ref[...] = p_out / jnp.maximum(nrm, 1e-12)


def _readout(h, ms1, ms2, batch_ids, rp, hp, rows=2000):
    q = rp["query"].reshape(HEADS, HEAD_DIM)
    eye = jnp.eye(HEADS, dtype=jnp.float32)
    qmat = (q[:, :, None] * eye[:, None, :]).reshape(HIDDEN, HEADS)
    r4 = jnp.repeat(eye, HEAD_DIM, axis=1)
    bid = batch_ids.reshape(N, 1).astype(jnp.int32)
    bid_t = batch_ids.reshape(1, N).astype(jnp.int32)
    full = lambda shape: pl.BlockSpec(shape, lambda i: (0, 0))
    row_h = pl.BlockSpec((rows, HIDDEN), lambda i: (i, 0))
    row_s = pl.BlockSpec((rows, HEADS), lambda i: (i, 0))
    row_b = pl.BlockSpec((rows, 1), lambda i: (i, 0))

    scores = pl.pallas_call(
        _scores_body,
        grid=(N // rows,),
        in_specs=[row_h, full((HIDDEN, HIDDEN)), full((1, HIDDEN)),
                  full((HIDDEN, HEADS))],
        out_specs=row_s,
        out_shape=jax.ShapeDtypeStruct((N, HEADS), jnp.float32),
    )(h, rp["key"]["W"], rp["key"]["b"].reshape(1, HIDDEN), qmat)

    attn = pl.pallas_call(
        _softmax_body,
        out_shape=jax.ShapeDtypeStruct((N, HEADS), jnp.float32),
    )(scores, bid_t)

    gspec = pl.BlockSpec((G, HIDDEN), lambda i: (0, 0))
    gemb, p1, p2 = pl.pallas_call(
        _pool_body,
        grid=(N // rows,),
        in_specs=[row_h, row_h, row_h, row_s, row_b,
                  full((HIDDEN, HIDDEN)), full((1, HIDDEN)),
                  full((HEADS, HIDDEN))],
        out_specs=[gspec, gspec, gspec],
        out_shape=[jax.ShapeDtypeStruct((G, HIDDEN), jnp.float32)] * 3,
    )(h, ms1, ms2, attn, bid,
      rp["value"]["W"], rp["value"]["b"].reshape(1, HIDDEN), r4)

    return pl.pallas_call(
        _head_body,
        out_shape=jax.ShapeDtypeStruct((G, OUT_DIM), jnp.float32),
    )(gemb, p1, p2,
      rp["out"]["W"], rp["out"]["b"].reshape(1, HIDDEN),
      rp["ng"].reshape(1, HIDDEN), rp["nb"].reshape(1, HIDDEN),
      hp["l1"]["W"], hp["l1"]["b"].reshape(1, HIDDEN),
      hp["l2"]["W"], hp["l2"]["b"].reshape(1, HIDDEN),
      hp["l3"]["W"], hp["l3"]["b"].reshape(1, OUT_DIM))


def _gine_aggr(h, ea, src, dst):
    mesh = plsc.VectorSubcoreMesh(core_axis_name="c", subcore_axis_name="s")

    @functools.partial(
        pl.kernel, mesh=mesh,
        out_type=jax.ShapeDtypeStruct((2, N, HIDDEN), jnp.float32),
        scratch_types=[
            pltpu.VMEM(((_CPW + 1) * _CH,), jnp.int32),
            pltpu.VMEM((_CH,), jnp.int32),
            pltpu.VMEM((_CH, HIDDEN), jnp.float32),
            pltpu.VMEM((_CH, HIDDEN), jnp.float32),
            pltpu.VMEM_SHARED((N, HIDDEN), jnp.float32),
            pltpu.SemaphoreType.DMA,
        ],
    )
    def k(h_hbm, ea_hbm, src_hbm, dst_hbm, z_hbm, out_hbm,
          srcall, dstv, rowsv, eav, aggr_sh, sem):
        c = lax.axis_index("c")
        s = lax.axis_index("s")
        wid = s * 2 + c
        roff = pl.multiple_of(s * _RPB, 8)
        @pl.when(s < 15)
        def _():
            pltpu.sync_copy(z_hbm.at[pl.ds(roff, _RPB)],
                            aggr_sh.at[pl.ds(roff, _RPB)])

        @pl.when(s == 15)
        def _():
            pltpu.sync_copy(z_hbm.at[pl.ds(roff, _RPB_LAST)],
                            aggr_sh.at[pl.ds(roff, _RPB_LAST)])

        plsc.subcore_barrier()
        nch = jnp.where(wid < _REM, _CPW + 1, _CPW)
        cbase = wid * _CPW + jnp.minimum(wid, _REM)
        pltpu.sync_copy(
            src_hbm.at[pl.ds(pl.multiple_of(cbase * _CH, _CH),
                             (_CPW + 1) * _CH)], srcall)

        def chunk(kk, carry):
            eoff = pl.multiple_of((cbase + kk) * _CH, _CH)
            pltpu.sync_copy(dst_hbm.at[pl.ds(eoff, _CH)], dstv)
            ioff = pl.multiple_of(kk * _CH, _CH)
            cp = pltpu.async_copy(
                h_hbm.at[srcall.at[pl.ds(ioff, _CH)]], rowsv, sem)
            pltpu.sync_copy(ea_hbm.at[pl.ds(eoff, _CH)], eav)
            cp.wait()

            def rbody(i, cr):
                for j in range(HIDDEN // _LANES):
                    sl = pl.ds(j * _LANES, _LANES)
                    rowsv[i, sl] = jnp.maximum(rowsv[i, sl] + eav[i, sl], 0.0)
                return cr

            lax.fori_loop(0, _CH, rbody, 0)
            pltpu.sync_copy(rowsv, aggr_sh.at[dstv], add=True)
            return carry

        lax.fori_loop(0, nch, chunk, 0)
        plsc.subcore_barrier()

        @pl.when(s < 15)
        def _():
            pltpu.sync_copy(aggr_sh.at[pl.ds(roff, _RPB)],
                            out_hbm.at[c, pl.ds(roff, _RPB)])

        @pl.when(s == 15)
        def _():
            pltpu.sync_copy(aggr_sh.at[pl.ds(roff, _RPB_LAST)],
                            out_hbm.at[c, pl.ds(roff, _RPB_LAST)])

    src_pad = jnp.concatenate([src, jnp.zeros((_CH * 8,), jnp.int32)])
    return k(h, ea, src_pad, dst, jnp.zeros((N, HIDDEN), jnp.float32))


def kernel(x, edge_attr, params, edge_index, batch_ids):
    p = params
    h = _mlp(x, p["node_enc"], rows=2000)
    ea = _mlp(edge_attr, p["edge_enc"], rows=4000)
    src = edge_index[0]
    dst = edge_index[1]
    ms = []
    for i in range(LAYERS):
        aggr2 = _gine_aggr(h, ea, src, dst)
        h, m = _conv(h, aggr2, p["convs"][i], p["norms"][i], p["ms"][i],
                     rows=2000)
        ms.append(m)
    return _readout(h, ms[0], ms[1], batch_ids, p["readout"], p["head"])


# speedup vs baseline: 1.8204x; 1.0690x over previous
import functools

import jax
import jax.numpy as jnp
from jax import lax
from jax.experimental import pallas as pl
from jax.experimental.pallas import tpu as pltpu
from jax.experimental.pallas import tpu_sc as plsc

N = 10000
E = 320000
NODE_DIM = 128
EDGE_DIM = 16
HIDDEN = 128
LAYERS = 2
OUT_DIM = 128
HEADS = 4
HEAD_DIM = HIDDEN // HEADS
G = 16

_LANES = 16
_CH = 128
_NCHUNKS = E // _CH
_NW = 32
_CPW = _NCHUNKS // _NW
_REM = _NCHUNKS - _CPW * _NW
_RPB = 632
_RPB_LAST = N - 15 * _RPB


def _mlp_body(x_ref, w1_ref, b1_ref, w2_ref, b2_ref, o_ref):
    t = jnp.maximum(x_ref[...] @ w1_ref[...] + b1_ref[...], 0.0)
    o_ref[...] = t @ w2_ref[...] + b2_ref[...]


def _mlp(x, p, rows):
    n, din = x.shape
    dh = p["l1"]["W"].shape[1]
    dout = p["l2"]["W"].shape[1]
    return pl.pallas_call(
        _mlp_body,
        grid=(n // rows,),
        in_specs=[
            pl.BlockSpec((rows, din), lambda i: (i, 0)),
            pl.BlockSpec((din, dh), lambda i: (0, 0)),
            pl.BlockSpec((1, dh), lambda i: (0, 0)),
            pl.BlockSpec((dh, dout), lambda i: (0, 0)),
            pl.BlockSpec((1, dout), lambda i: (0, 0)),
        ],
        out_specs=pl.BlockSpec((rows, dout), lambda i: (i, 0)),
        out_shape=jax.ShapeDtypeStruct((n, dout), jnp.float32),
    )(x, p["l1"]["W"], p["l1"]["b"].reshape(1, dh),
      p["l2"]["W"], p["l2"]["b"].reshape(1, dout))


def _conv_body(h_ref, a0_ref, a1_ref, w1_ref, b1_ref, w2_ref, b2_ref,
               g_ref, bn_ref, wm_ref, bm_ref, hn_ref, ms_ref):
    h = h_ref[...]
    a = h + a0_ref[0] + a1_ref[0]
    t = jnp.maximum(a @ w1_ref[...] + b1_ref[...], 0.0)
    o = jnp.maximum(t @ w2_ref[...] + b2_ref[...], 0.0)
    o = o + h
    m = jnp.mean(o, axis=-1, keepdims=True)
    v = jnp.mean((o - m) ** 2, axis=-1, keepdims=True)
    hn = (o - m) / jnp.sqrt(v + 1e-5) * g_ref[...] + bn_ref[...]
    hn_ref[...] = hn
    ms_ref[...] = hn @ wm_ref[...] + bm_ref[...]


def _conv(h, aggr2, cp, np_, mp, rows):
    full = lambda shape: pl.BlockSpec(shape, lambda i: (0, 0))
    row_spec = pl.BlockSpec((rows, HIDDEN), lambda i: (i, 0))
    hn, ms = pl.pallas_call(
        _conv_body,
        grid=(N // rows,),
        in_specs=[
            row_spec,
            pl.BlockSpec((1, rows, HIDDEN), lambda i: (0, i, 0)),
            pl.BlockSpec((1, rows, HIDDEN), lambda i: (1, i, 0)),
            full((HIDDEN, HIDDEN)), full((1, HIDDEN)),
            full((HIDDEN, HIDDEN)), full((1, HIDDEN)),
            full((1, HIDDEN)), full((1, HIDDEN)),
            full((HIDDEN, HIDDEN)), full((1, HIDDEN)),
        ],
        out_specs=[row_spec, row_spec],
        out_shape=[jax.ShapeDtypeStruct((N, HIDDEN), jnp.float32),
                   jax.ShapeDtypeStruct((N, HIDDEN), jnp.float32)],
    )(h, aggr2, aggr2,
      cp["l1"]["W"], cp["l1"]["b"].reshape(1, HIDDEN),
      cp["l2"]["W"], cp["l2"]["b"].reshape(1, HIDDEN),
      np_["g"].reshape(1, HIDDEN), np_["b"].reshape(1, HIDDEN),
      mp["W"], mp["b"].reshape(1, HIDDEN))
    return hn, ms


_CONTRACT0 = (((0,), (0,)), ((), ()))


def _onehot(bid):
    return (bid == lax.broadcasted_iota(jnp.int32, (1, G), 1)
            ).astype(jnp.float32)


def _scores_body(h_ref, wk_ref, bk_ref, qm_ref, s_ref):
    k = h_ref[...] @ wk_ref[...] + bk_ref[...]
    s_ref[...] = (k @ qm_ref[...]) * (1.0 / jnp.sqrt(jnp.float32(HEAD_DIM)))


def _softmax_body(s_ref, bid_ref, a_ref):
    scores = s_ref[...].T
    p_t = (bid_ref[...] == lax.broadcasted_iota(jnp.int32, (G, 1), 0)
           ).astype(jnp.float32)
    smax_cols = []
    for g in range(G):
        mask = p_t[g:g + 1, :] > 0.0
        sg = jnp.max(jnp.where(mask, scores, -jnp.inf), axis=1, keepdims=True)
        smax_cols.append(sg)
    smax = jnp.concatenate(smax_cols, axis=1)
    smax = jnp.where(smax > -1e30, smax, 0.0)
    e = jnp.exp(scores - smax @ p_t)
    contract1 = (((1,), (1,)), ((), ()))
    denom = lax.dot_general(e, p_t, contract1)
    a_ref[...] = (e / jnp.maximum(denom @ p_t, 1e-12)).T


def _pool_body(h_ref, ms1_ref, ms2_ref, attn_ref, bid_ref,
               wv_ref, bv_ref, r4_ref, gemb_ref, p1_ref, p2_ref):
    i = pl.program_id(0)
    p_onehot = _onehot(bid_ref[...])
    v = h_ref[...] @ wv_ref[...] + bv_ref[...]
    wvw = (attn_ref[...] @ r4_ref[...]) * v

    @pl.when(i == 0)
    def _():
        gemb_ref[...] = jnp.zeros_like(gemb_ref)
        p1_ref[...] = jnp.zeros_like(p1_ref)
        p2_ref[...] = jnp.zeros_like(p2_ref)

    gemb_ref[...] += lax.dot_general(p_onehot, wvw, _CONTRACT0)
    p1_ref[...] += lax.dot_general(p_onehot, ms1_ref[...], _CONTRACT0)
    p2_ref[...] += lax.dot_general(p_onehot, ms2_ref[...], _CONTRACT0)


def _head_body(gemb_ref, p1_ref, p2_ref, wo_ref, bo_ref, ng_ref, nb_ref,
               w1_ref, b1_ref, w2_ref, b2_ref, w3_ref, b3_ref, o_ref):
    gemb = gemb_ref[...] @ wo_ref[...] + bo_ref[...]
    m = jnp.mean(gemb, axis=-1, keepdims=True)
    var = jnp.mean((gemb - m) ** 2, axis=-1, keepdims=True)
    gemb = (gemb - m) / jnp.sqrt(var + 1e-5) * ng_ref[...] + nb_ref[...]
    cat = jnp.concatenate([gemb, p1_ref[...], p2_ref[...]], axis=-1)
    t = jnp.maximum(cat @ w1_ref[...] + b1_ref[...], 0.0)
    t = jnp.maximum(t @ w2_ref[...] + b2_ref[...], 0.0)
    p_out = t @ w3_ref[...] + b3_ref[...]
    nrm = jnp.sqrt(jnp.sum(p_out * p_out, axis=-1, keepdims=True))
    o_ref[...] = p_out / jnp.maximum(nrm, 1e-12)


def _readout(h, ms1, ms2, batch_ids, rp, hp, rows=2000):
    q = rp["query"].reshape(HEADS, HEAD_DIM)
    eye = jnp.eye(HEADS, dtype=jnp.float32)
    qmat = (q[:, :, None] * eye[:, None, :]).reshape(HIDDEN, HEADS)
    r4 = jnp.repeat(eye, HEAD_DIM, axis=1)
    bid = batch_ids.reshape(N, 1).astype(jnp.int32)
    bid_t = batch_ids.reshape(1, N).astype(jnp.int32)
    full = lambda shape: pl.BlockSpec(shape, lambda i: (0, 0))
    row_h = pl.BlockSpec((rows, HIDDEN), lambda i: (i, 0))
    row_s = pl.BlockSpec((rows, HEADS), lambda i: (i, 0))
    row_b = pl.BlockSpec((rows, 1), lambda i: (i, 0))

    scores = pl.pallas_call(
        _scores_body,
        grid=(N // rows,),
        in_specs=[row_h, full((HIDDEN, HIDDEN)), full((1, HIDDEN)),
                  full((HIDDEN, HEADS))],
        out_specs=row_s,
        out_shape=jax.ShapeDtypeStruct((N, HEADS), jnp.float32),
    )(h, rp["key"]["W"], rp["key"]["b"].reshape(1, HIDDEN), qmat)

    attn = pl.pallas_call(
        _softmax_body,
        out_shape=jax.ShapeDtypeStruct((N, HEADS), jnp.float32),
    )(scores, bid_t)

    gspec = pl.BlockSpec((G, HIDDEN), lambda i: (0, 0))
    gemb, p1, p2 = pl.pallas_call(
        _pool_body,
        grid=(N // rows,),
        in_specs=[row_h, row_h, row_h, row_s, row_b,
                  full((HIDDEN, HIDDEN)), full((1, HIDDEN)),
                  full((HEADS, HIDDEN))],
        out_specs=[gspec, gspec, gspec],
        out_shape=[jax.ShapeDtypeStruct((G, HIDDEN), jnp.float32)] * 3,
    )(h, ms1, ms2, attn, bid,
      rp["value"]["W"], rp["value"]["b"].reshape(1, HIDDEN), r4)

    return pl.pallas_call(
        _head_body,
        out_shape=jax.ShapeDtypeStruct((G, OUT_DIM), jnp.float32),
    )(gemb, p1, p2,
      rp["out"]["W"], rp["out"]["b"].reshape(1, HIDDEN),
      rp["ng"].reshape(1, HIDDEN), rp["nb"].reshape(1, HIDDEN),
      hp["l1"]["W"], hp["l1"]["b"].reshape(1, HIDDEN),
      hp["l2"]["W"], hp["l2"]["b"].reshape(1, HIDDEN),
      hp["l3"]["W"], hp["l3"]["b"].reshape(1, OUT_DIM))


def _gine_aggr(h, ea, src, dst):
    mesh = plsc.VectorSubcoreMesh(core_axis_name="c", subcore_axis_name="s")

    @functools.partial(
        pl.kernel, mesh=mesh,
        out_type=jax.ShapeDtypeStruct((2, N, HIDDEN), jnp.float32),
        scratch_types=[
            pltpu.VMEM(((_CPW + 1) * _CH,), jnp.int32),
            pltpu.VMEM((_CH,), jnp.int32),
            pltpu.VMEM((_CH, HIDDEN), jnp.float32),
            pltpu.VMEM((_CH, HIDDEN), jnp.float32),
            pltpu.VMEM_SHARED((N, HIDDEN), jnp.float32),
            pltpu.SemaphoreType.DMA,
        ],
    )
    def k(h_hbm, ea_hbm, src_hbm, dst_hbm, z_hbm, out_hbm,
          srcall, dstv, rowsv, eav, aggr_sh, sem):
        c = lax.axis_index("c")
        s = lax.axis_index("s")
        wid = s * 2 + c
        roff = pl.multiple_of(s * _RPB, 8)
        @pl.when(s < 15)
        def _():
            pltpu.sync_copy(z_hbm.at[pl.ds(roff, _RPB)],
                            aggr_sh.at[pl.ds(roff, _RPB)])

        @pl.when(s == 15)
        def _():
            pltpu.sync_copy(z_hbm.at[pl.ds(roff, _RPB_LAST)],
                            aggr_sh.at[pl.ds(roff, _RPB_LAST)])

        plsc.subcore_barrier()
        nch = jnp.where(wid < _REM, _CPW + 1, _CPW)
        cbase = wid * _CPW + jnp.minimum(wid, _REM)
        pltpu.sync_copy(
            src_hbm.at[pl.ds(pl.multiple_of(cbase * _CH, _CH),
                             (_CPW + 1) * _CH)], srcall)

        def chunk(kk, carry):
            eoff = pl.multiple_of((cbase + kk) * _CH, _CH)
            ioff = pl.multiple_of(kk * _CH, _CH)
            cp = pltpu.async_copy(
                h_hbm.at[srcall.at[pl.ds(ioff, _CH)]], rowsv, sem)
            pltpu.sync_copy(dst_hbm.at[pl.ds(eoff, _CH)], dstv)
            pltpu.sync_copy(ea_hbm.at[pl.ds(eoff, _CH)], eav)
            cp.wait()

            def rbody(i, cr):
                for j in range(HIDDEN // _LANES):
                    sl = pl.ds(j * _LANES, _LANES)
                    rowsv[i, sl] = jnp.maximum(rowsv[i, sl] + eav[i, sl], 0.0)
                return cr

            lax.fori_loop(0, _CH, rbody, 0)
            pltpu.sync_copy(rowsv, aggr_sh.at[dstv], add=True)
            return carry

        lax.fori_loop(0, nch, chunk, 0)
        plsc.subcore_barrier()

        @pl.when(s < 15)
        def _():
            pltpu.sync_copy(aggr_sh.at[pl.ds(roff, _RPB)],
                            out_hbm.at[c, pl.ds(roff, _RPB)])

        @pl.when(s == 15)
        def _():
            pltpu.sync_copy(aggr_sh.at[pl.ds(roff, _RPB_LAST)],
                            out_hbm.at[c, pl.ds(roff, _RPB_LAST)])

    src_pad = jnp.concatenate([src, jnp.zeros((_CH * 8,), jnp.int32)])
    return k(h, ea, src_pad, dst, jnp.zeros((N, HIDDEN), jnp.float32))


def kernel(x, edge_attr, params, edge_index, batch_ids):
    p = params
    h = _mlp(x, p["node_enc"], rows=2000)
    ea = _mlp(edge_attr, p["edge_enc"], rows=4000)
    src = edge_index[0]
    dst = edge_index[1]
    ms = []
    for i in range(LAYERS):
        aggr2 = _gine_aggr(h, ea, src, dst)
        h, m = _conv(h, aggr2, p["convs"][i], p["norms"][i], p["ms"][i],
                     rows=2000)
        ms.append(m)
    return _readout(h, ms[0], ms[1], batch_ids, p["readout"], p["head"])


# speedup vs baseline: 2.0032x; 1.1004x over previous
import functools

import jax
import jax.numpy as jnp
from jax import lax
from jax.experimental import pallas as pl
from jax.experimental.pallas import tpu as pltpu
from jax.experimental.pallas import tpu_sc as plsc

N = 10000
E = 320000
NODE_DIM = 128
EDGE_DIM = 16
HIDDEN = 128
LAYERS = 2
OUT_DIM = 128
HEADS = 4
HEAD_DIM = HIDDEN // HEADS
G = 16

_LANES = 16
_CH = 128
_NCHUNKS = E // _CH
_NW = 32
_CPW = _NCHUNKS // _NW
_REM = _NCHUNKS - _CPW * _NW
_RPB = 632
_RPB_LAST = N - 15 * _RPB


def _mlp_body(x_ref, w1_ref, b1_ref, w2_ref, b2_ref, o_ref):
    t = jnp.maximum(x_ref[...] @ w1_ref[...] + b1_ref[...], 0.0)
    o_ref[...] = t @ w2_ref[...] + b2_ref[...]


def _mlp(x, p, rows):
    n, din = x.shape
    dh = p["l1"]["W"].shape[1]
    dout = p["l2"]["W"].shape[1]
    return pl.pallas_call(
        _mlp_body,
        grid=(n // rows,),
        in_specs=[
            pl.BlockSpec((rows, din), lambda i: (i, 0)),
            pl.BlockSpec((din, dh), lambda i: (0, 0)),
            pl.BlockSpec((1, dh), lambda i: (0, 0)),
            pl.BlockSpec((dh, dout), lambda i: (0, 0)),
            pl.BlockSpec((1, dout), lambda i: (0, 0)),
        ],
        out_specs=pl.BlockSpec((rows, dout), lambda i: (i, 0)),
        out_shape=jax.ShapeDtypeStruct((n, dout), jnp.float32),
    )(x, p["l1"]["W"], p["l1"]["b"].reshape(1, dh),
      p["l2"]["W"], p["l2"]["b"].reshape(1, dout))


def _conv_body(h_ref, a0_ref, a1_ref, w1_ref, b1_ref, w2_ref, b2_ref,
               g_ref, bn_ref, wm_ref, bm_ref, hn_ref, ms_ref):
    h = h_ref[...]
    a = h + a0_ref[0] + a1_ref[0]
    t = jnp.maximum(a @ w1_ref[...] + b1_ref[...], 0.0)
    o = jnp.maximum(t @ w2_ref[...] + b2_ref[...], 0.0)
    o = o + h
    m = jnp.mean(o, axis=-1, keepdims=True)
    v = jnp.mean((o - m) ** 2, axis=-1, keepdims=True)
    hn = (o - m) / jnp.sqrt(v + 1e-5) * g_ref[...] + bn_ref[...]
    hn_ref[...] = hn
    ms_ref[...] = hn @ wm_ref[...] + bm_ref[...]


def _conv(h, aggr2, cp, np_, mp, rows):
    full = lambda shape: pl.BlockSpec(shape, lambda i: (0, 0))
    row_spec = pl.BlockSpec((rows, HIDDEN), lambda i: (i, 0))
    hn, ms = pl.pallas_call(
        _conv_body,
        grid=(N // rows,),
        in_specs=[
            row_spec,
            pl.BlockSpec((1, rows, HIDDEN), lambda i: (0, i, 0)),
            pl.BlockSpec((1, rows, HIDDEN), lambda i: (1, i, 0)),
            full((HIDDEN, HIDDEN)), full((1, HIDDEN)),
            full((HIDDEN, HIDDEN)), full((1, HIDDEN)),
            full((1, HIDDEN)), full((1, HIDDEN)),
            full((HIDDEN, HIDDEN)), full((1, HIDDEN)),
        ],
        out_specs=[row_spec, row_spec],
        out_shape=[jax.ShapeDtypeStruct((N, HIDDEN), jnp.float32),
                   jax.ShapeDtypeStruct((N, HIDDEN), jnp.float32)],
    )(h, aggr2, aggr2,
      cp["l1"]["W"], cp["l1"]["b"].reshape(1, HIDDEN),
      cp["l2"]["W"], cp["l2"]["b"].reshape(1, HIDDEN),
      np_["g"].reshape(1, HIDDEN), np_["b"].reshape(1, HIDDEN),
      mp["W"], mp["b"].reshape(1, HIDDEN))
    return hn, ms


_CONTRACT0 = (((0,), (0,)), ((), ()))


def _onehot(bid):
    return (bid == lax.broadcasted_iota(jnp.int32, (1, G), 1)
            ).astype(jnp.float32)


def _scores_body(h_ref, wk_ref, bk_ref, qm_ref, s_ref):
    k = h_ref[...] @ wk_ref[...] + bk_ref[...]
    s_ref[...] = (k @ qm_ref[...]) * (1.0 / jnp.sqrt(jnp.float32(HEAD_DIM)))


def _softmax_body(s_ref, bid_ref, a_ref):
    scores = s_ref[...].T
    p_t = (bid_ref[...] == lax.broadcasted_iota(jnp.int32, (G, 1), 0)
           ).astype(jnp.float32)
    smax_cols = []
    for g in range(G):
        mask = p_t[g:g + 1, :] > 0.0
        sg = jnp.max(jnp.where(mask, scores, -jnp.inf), axis=1, keepdims=True)
        smax_cols.append(sg)
    smax = jnp.concatenate(smax_cols, axis=1)
    smax = jnp.where(smax > -1e30, smax, 0.0)
    e = jnp.exp(scores - smax @ p_t)
    contract1 = (((1,), (1,)), ((), ()))
    denom = lax.dot_general(e, p_t, contract1)
    a_ref[...] = (e / jnp.maximum(denom @ p_t, 1e-12)).T


def _pool_body(h_ref, ms1_ref, ms2_ref, attn_ref, bid_ref,
               wv_ref, bv_ref, r4_ref, gemb_ref, p1_ref, p2_ref):
    i = pl.program_id(0)
    p_onehot = _onehot(bid_ref[...])
    v = h_ref[...] @ wv_ref[...] + bv_ref[...]
    wvw = (attn_ref[...] @ r4_ref[...]) * v

    @pl.when(i == 0)
    def _():
        gemb_ref[...] = jnp.zeros_like(gemb_ref)
        p1_ref[...] = jnp.zeros_like(p1_ref)
        p2_ref[...] = jnp.zeros_like(p2_ref)

    gemb_ref[...] += lax.dot_general(p_onehot, wvw, _CONTRACT0)
    p1_ref[...] += lax.dot_general(p_onehot, ms1_ref[...], _CONTRACT0)
    p2_ref[...] += lax.dot_general(p_onehot, ms2_ref[...], _CONTRACT0)


def _head_body(gemb_ref, p1_ref, p2_ref, wo_ref, bo_ref, ng_ref, nb_ref,
               w1_ref, b1_ref, w2_ref, b2_ref, w3_ref, b3_ref, o_ref):
    gemb = gemb_ref[...] @ wo_ref[...] + bo_ref[...]
    m = jnp.mean(gemb, axis=-1, keepdims=True)
    var = jnp.mean((gemb - m) ** 2, axis=-1, keepdims=True)
    gemb = (gemb - m) / jnp.sqrt(var + 1e-5) * ng_ref[...] + nb_ref[...]
    cat = jnp.concatenate([gemb, p1_ref[...], p2_ref[...]], axis=-1)
    t = jnp.maximum(cat @ w1_ref[...] + b1_ref[...], 0.0)
    t = jnp.maximum(t @ w2_ref[...] + b2_ref[...], 0.0)
    p_out = t @ w3_ref[...] + b3_ref[...]
    nrm = jnp.sqrt(jnp.sum(p_out * p_out, axis=-1, keepdims=True))
    o_ref[...] = p_out / jnp.maximum(nrm, 1e-12)


def _readout(h, ms1, ms2, batch_ids, rp, hp, rows=2000):
    q = rp["query"].reshape(HEADS, HEAD_DIM)
    eye = jnp.eye(HEADS, dtype=jnp.float32)
    qmat = (q[:, :, None] * eye[:, None, :]).reshape(HIDDEN, HEADS)
    r4 = jnp.repeat(eye, HEAD_DIM, axis=1)
    bid = batch_ids.reshape(N, 1).astype(jnp.int32)
    bid_t = batch_ids.reshape(1, N).astype(jnp.int32)
    full = lambda shape: pl.BlockSpec(shape, lambda i: (0, 0))
    row_h = pl.BlockSpec((rows, HIDDEN), lambda i: (i, 0))
    row_s = pl.BlockSpec((rows, HEADS), lambda i: (i, 0))
    row_b = pl.BlockSpec((rows, 1), lambda i: (i, 0))

    scores = pl.pallas_call(
        _scores_body,
        grid=(N // rows,),
        in_specs=[row_h, full((HIDDEN, HIDDEN)), full((1, HIDDEN)),
                  full((HIDDEN, HEADS))],
        out_specs=row_s,
        out_shape=jax.ShapeDtypeStruct((N, HEADS), jnp.float32),
    )(h, rp["key"]["W"], rp["key"]["b"].reshape(1, HIDDEN), qmat)

    attn = pl.pallas_call(
        _softmax_body,
        out_shape=jax.ShapeDtypeStruct((N, HEADS), jnp.float32),
    )(scores, bid_t)

    gspec = pl.BlockSpec((G, HIDDEN), lambda i: (0, 0))
    gemb, p1, p2 = pl.pallas_call(
        _pool_body,
        grid=(N // rows,),
        in_specs=[row_h, row_h, row_h, row_s, row_b,
                  full((HIDDEN, HIDDEN)), full((1, HIDDEN)),
                  full((HEADS, HIDDEN))],
        out_specs=[gspec, gspec, gspec],
        out_shape=[jax.ShapeDtypeStruct((G, HIDDEN), jnp.float32)] * 3,
    )(h, ms1, ms2, attn, bid,
      rp["value"]["W"], rp["value"]["b"].reshape(1, HIDDEN), r4)

    return pl.pallas_call(
        _head_body,
        out_shape=jax.ShapeDtypeStruct((G, OUT_DIM), jnp.float32),
    )(gemb, p1, p2,
      rp["out"]["W"], rp["out"]["b"].reshape(1, HIDDEN),
      rp["ng"].reshape(1, HIDDEN), rp["nb"].reshape(1, HIDDEN),
      hp["l1"]["W"], hp["l1"]["b"].reshape(1, HIDDEN),
      hp["l2"]["W"], hp["l2"]["b"].reshape(1, HIDDEN),
      hp["l3"]["W"], hp["l3"]["b"].reshape(1, OUT_DIM))


def _gine_aggr(h, ea, src, dst):
    mesh = plsc.VectorSubcoreMesh(core_axis_name="c", subcore_axis_name="s")

    @functools.partial(
        pl.kernel, mesh=mesh,
        out_type=jax.ShapeDtypeStruct((2, N, HIDDEN), jnp.float32),
        scratch_types=[
            pltpu.VMEM(((_CPW + 1) * _CH,), jnp.int32),
            pltpu.VMEM((2, _CH), jnp.int32),
            pltpu.VMEM((_CH, HIDDEN), jnp.float32),
            pltpu.VMEM((_CH, HIDDEN), jnp.float32),
            pltpu.VMEM_SHARED((N, HIDDEN), jnp.float32),
            pltpu.SemaphoreType.DMA,
            pltpu.SemaphoreType.DMA,
        ],
    )
    def k(h_hbm, ea_hbm, src_hbm, dst_hbm, z_hbm, out_hbm,
          srcall, dstv, rowsv, eav, aggr_sh, semg, sems):
        c = lax.axis_index("c")
        s = lax.axis_index("s")
        wid = s * 2 + c
        roff = pl.multiple_of(s * _RPB, 8)
        @pl.when(s < 15)
        def _():
            pltpu.sync_copy(z_hbm.at[pl.ds(roff, _RPB)],
                            aggr_sh.at[pl.ds(roff, _RPB)])

        @pl.when(s == 15)
        def _():
            pltpu.sync_copy(z_hbm.at[pl.ds(roff, _RPB_LAST)],
                            aggr_sh.at[pl.ds(roff, _RPB_LAST)])

        plsc.subcore_barrier()
        nch = jnp.where(wid < _REM, _CPW + 1, _CPW)
        cbase = wid * _CPW + jnp.minimum(wid, _REM)
        pltpu.sync_copy(
            src_hbm.at[pl.ds(pl.multiple_of(cbase * _CH, _CH),
                             (_CPW + 1) * _CH)], srcall)

        def gref(kk):
            ioff = pl.multiple_of(kk * _CH, _CH)
            return h_hbm.at[srcall.at[pl.ds(ioff, _CH)]]

        pltpu.async_copy(gref(0), rowsv, semg)

        def chunk(kk, carry):
            b = lax.rem(kk, 2)
            eoff = pl.multiple_of((cbase + kk) * _CH, _CH)
            pltpu.sync_copy(dst_hbm.at[pl.ds(eoff, _CH)], dstv.at[b])

            @pl.when(kk >= 1)
            def _():
                pltpu.make_async_copy(
                    eav, aggr_sh.at[dstv.at[1 - b]], sems).wait()

            pltpu.sync_copy(ea_hbm.at[pl.ds(eoff, _CH)], eav)
            pltpu.make_async_copy(gref(kk), rowsv, semg).wait()

            def rbody(i, cr):
                for j in range(HIDDEN // _LANES):
                    sl = pl.ds(j * _LANES, _LANES)
                    eav[i, sl] = jnp.maximum(rowsv[i, sl] + eav[i, sl], 0.0)
                return cr

            lax.fori_loop(0, _CH, rbody, 0)
            pltpu.async_copy(gref(jnp.minimum(kk + 1, nch - 1)), rowsv, semg)
            pltpu.async_copy(eav, aggr_sh.at[dstv.at[b]], sems, add=True)
            return carry

        lax.fori_loop(0, nch, chunk, 0)
        pltpu.make_async_copy(gref(nch - 1), rowsv, semg).wait()
        pltpu.make_async_copy(
            eav, aggr_sh.at[dstv.at[lax.rem(nch - 1, 2)]], sems).wait()
        plsc.subcore_barrier()

        @pl.when(s < 15)
        def _():
            pltpu.sync_copy(aggr_sh.at[pl.ds(roff, _RPB)],
                            out_hbm.at[c, pl.ds(roff, _RPB)])

        @pl.when(s == 15)
        def _():
            pltpu.sync_copy(aggr_sh.at[pl.ds(roff, _RPB_LAST)],
                            out_hbm.at[c, pl.ds(roff, _RPB_LAST)])

    src_pad = jnp.concatenate([src, jnp.zeros((_CH * 8,), jnp.int32)])
    return k(h, ea, src_pad, dst, jnp.zeros((N, HIDDEN), jnp.float32))


def kernel(x, edge_attr, params, edge_index, batch_ids):
    p = params
    h = _mlp(x, p["node_enc"], rows=2000)
    ea = _mlp(edge_attr, p["edge_enc"], rows=4000)
    src = edge_index[0]
    dst = edge_index[1]
    ms = []
    for i in range(LAYERS):
        aggr2 = _gine_aggr(h, ea, src, dst)
        h, m = _conv(h, aggr2, p["convs"][i], p["norms"][i], p["ms"][i],
                     rows=2000)
        ms.append(m)
    return _readout(h, ms[0], ms[1], batch_ids, p["readout"], p["head"])
